# initial kernel scaffold (unmeasured)
import jax
import jax.numpy as jnp
from jax import lax
from jax.experimental import pallas as pl
from jax.experimental.pallas import tpu as pltpu

N_DEV = 4
SQ = 2048
SKV = 2048
DM = 1024
DH = 128
HPD = 8
SCALE = 0.08838834764831843
QTILE = 512
NTILES = SQ // QTILE
NEG = -1e9


def kernel(x, Wq, K_ext, V_ext, Wo):
    def body(x_ref, wq_hbm, k_hbm, v_hbm, wo_hbm, out_ref,
             comm_ref, q_ref, ctx_ref, kg_ref, vg_ref, mask_ref,
             send_sems, recv_sems, sem_wq, sem_wo, sem_k, sem_v):
        my = lax.axis_index("i")
        left = lax.rem(my + N_DEV - 1, N_DEV)
        right = lax.rem(my + 1, N_DEV)

        cp_wq = pltpu.make_async_copy(wq_hbm, comm_ref.at[0, 0], sem_wq)
        cp_wo = pltpu.make_async_copy(wo_hbm, comm_ref.at[0, 1], sem_wo)
        cp_wq.start()
        cp_wo.start()

        r = lax.broadcasted_iota(jnp.int32, (QTILE, SKV), 0)
        c = lax.broadcasted_iota(jnp.int32, (QTILE, SKV), 1)
        mask_ref[...] = jnp.where(
            (r // 64) % 4 == (c // 64) % 4, 0.0, NEG
        ).astype(jnp.float32)

        barrier_sem = pltpu.get_barrier_semaphore()
        for nbr in (left, right):
            pl.semaphore_signal(
                barrier_sem, inc=1,
                device_id=(nbr,), device_id_type=pl.DeviceIdType.MESH,
            )
        pl.semaphore_wait(barrier_sem, 2)

        cp_wq.wait()
        cp_wo.wait()

        x2d = x_ref[0]

        for h in range(N_DEV):
            if h < N_DEV - 1:
                rdma = pltpu.make_async_remote_copy(
                    src_ref=comm_ref.at[h],
                    dst_ref=comm_ref.at[h + 1],
                    send_sem=send_sems.at[h],
                    recv_sem=recv_sems.at[h + 1],
                    device_id=(right,),
                    device_id_type=pl.DeviceIdType.MESH,
                )
                rdma.start()

            j = lax.rem(my - h + N_DEV, N_DEV)

            cp_k = pltpu.make_async_copy(
                k_hbm.at[my, :, pl.ds(j * HPD, HPD), :], kg_ref, sem_k)
            cp_v = pltpu.make_async_copy(
                v_hbm.at[my, :, pl.ds(j * HPD, HPD), :], vg_ref, sem_v)
            cp_k.start()
            cp_v.start()

            q_ref[...] = jnp.dot(
                x2d, comm_ref[h, 0], preferred_element_type=jnp.float32)

            cp_k.wait()
            cp_v.wait()

            for hh in range(HPD):
                kh = kg_ref[:, hh, :]
                vh = vg_ref[:, hh, :]
                for t in range(NTILES):
                    qt = q_ref[t * QTILE:(t + 1) * QTILE,
                               hh * DH:(hh + 1) * DH]
                    s = lax.dot_general(
                        qt, kh,
                        dimension_numbers=(((1,), (1,)), ((), ())),
                        preferred_element_type=jnp.float32,
                    ) * SCALE + mask_ref[...]
                    m = jnp.max(s, axis=-1, keepdims=True)
                    w = jnp.exp(s - m)
                    d = jnp.sum(w, axis=-1, keepdims=True)
                    ctx_ref[t * QTILE:(t + 1) * QTILE,
                            hh * DH:(hh + 1) * DH] = jnp.dot(
                        w / d, vh, preferred_element_type=jnp.float32)

            contrib = jnp.dot(
                ctx_ref[...], comm_ref[h, 1],
                preferred_element_type=jnp.float32)
            if h == 0:
                out_ref[0] = contrib
            else:
                out_ref[0] = out_ref[0] + contrib

            if h < N_DEV - 1:
                rdma.wait()

    return pl.pallas_call(
        body,
        out_shape=jax.ShapeDtypeStruct((1, SQ, DM), jnp.float32),
        in_specs=[
            pl.BlockSpec(memory_space=pltpu.VMEM),
            pl.BlockSpec(memory_space=pltpu.ANY),
            pl.BlockSpec(memory_space=pltpu.ANY),
            pl.BlockSpec(memory_space=pltpu.ANY),
            pl.BlockSpec(memory_space=pltpu.ANY),
        ],
        out_specs=pl.BlockSpec(memory_space=pltpu.VMEM),
        scratch_shapes=[
            pltpu.VMEM((N_DEV, 2, DM, DM), jnp.float32),
            pltpu.VMEM((SQ, HPD * DH), jnp.float32),
            pltpu.VMEM((SQ, HPD * DH), jnp.float32),
            pltpu.VMEM((SKV, HPD, DH), jnp.float32),
            pltpu.VMEM((SKV, HPD, DH), jnp.float32),
            pltpu.VMEM((QTILE, SKV), jnp.float32),
            pltpu.SemaphoreType.DMA((N_DEV,)),
            pltpu.SemaphoreType.DMA((N_DEV,)),
            pltpu.SemaphoreType.DMA,
            pltpu.SemaphoreType.DMA,
            pltpu.SemaphoreType.DMA,
            pltpu.SemaphoreType.DMA,
        ],
        compiler_params=pltpu.CompilerParams(collective_id=0),
    )(x, Wq, K_ext, V_ext, Wo)


# baseline (device time: 516484 ns/iter reference)
import jax
import jax.numpy as jnp
from jax import lax
from jax.experimental import pallas as pl
from jax.experimental.pallas import tpu as pltpu

N_DEV = 4
SQ = 2048
SKV = 2048
DM = 1024
DH = 128
HPD = 8
SCALE = 0.08838834764831843
QTILE = 512
NTILES = SQ // QTILE


def kernel(x, Wq, K_ext, V_ext, Wo):
    def body(x_hbm, wq_hbm, k_hbm, v_hbm, wo_hbm, out_ref,
             comm_ref, x_ref, q_ref, ctx_ref, kg_ref, vg_ref,
             stage_ref, mask_ref,
             send_sems, recv_sems, k_sems, v_sems, sem_stage):
        my = lax.axis_index("i")
        left = lax.rem(my + N_DEV - 1, N_DEV)
        right = lax.rem(my + 1, N_DEV)

        def stage(src, dst):
            cp = pltpu.make_async_copy(src, stage_ref, sem_stage)
            cp.start()
            cp.wait()
            dst[...] = stage_ref[...].astype(jnp.bfloat16)

        stage(wq_hbm, comm_ref.at[0, 0])
        stage(wo_hbm, comm_ref.at[0, 1])
        stage(x_hbm.at[0, pl.ds(0, DM), :], x_ref.at[pl.ds(0, DM)])
        stage(x_hbm.at[0, pl.ds(DM, DM), :], x_ref.at[pl.ds(DM, DM)])

        r = lax.broadcasted_iota(jnp.int32, (QTILE, SKV), 0)
        c = lax.broadcasted_iota(jnp.int32, (QTILE, SKV), 1)
        mask_ref[...] = jnp.where(
            (r // 64) % 4 == (c // 64) % 4, 1.0, 0.0
        ).astype(jnp.bfloat16)

        barrier_sem = pltpu.get_barrier_semaphore()
        for nbr in (left, right):
            pl.semaphore_signal(
                barrier_sem, inc=1,
                device_id=(nbr,), device_id_type=pl.DeviceIdType.MESH,
            )
        pl.semaphore_wait(barrier_sem, 2)

        def fetch_kv(j, hh):
            p = hh % 2
            cp_k = pltpu.make_async_copy(
                k_hbm.at[my, :, j * HPD + hh, :], kg_ref.at[p], k_sems.at[p])
            cp_v = pltpu.make_async_copy(
                v_hbm.at[my, :, j * HPD + hh, :], vg_ref.at[p], v_sems.at[p])
            cp_k.start()
            cp_v.start()
            return cp_k, cp_v

        for h in range(N_DEV):
            if h < N_DEV - 1:
                rdma = pltpu.make_async_remote_copy(
                    src_ref=comm_ref.at[h],
                    dst_ref=comm_ref.at[h + 1],
                    send_sem=send_sems.at[h],
                    recv_sem=recv_sems.at[h + 1],
                    device_id=(right,),
                    device_id_type=pl.DeviceIdType.MESH,
                )
                rdma.start()

            j = lax.rem(my - h + N_DEV, N_DEV)

            pending = fetch_kv(j, 0)
            q_ref[...] = jnp.dot(
                x_ref[...], comm_ref[h, 0],
                preferred_element_type=jnp.float32,
            ).astype(jnp.bfloat16)

            for hh in range(HPD):
                p = hh % 2
                pending[0].wait()
                pending[1].wait()
                if hh < HPD - 1:
                    pending = fetch_kv(j, hh + 1)
                kh = kg_ref[p].astype(jnp.bfloat16)
                vh = vg_ref[p].astype(jnp.bfloat16)

                def tile_body(t, _, hh=hh, kh=kh, vh=vh):
                    row = t * QTILE
                    qt = q_ref[pl.ds(row, QTILE), hh * DH:(hh + 1) * DH]
                    s = lax.dot_general(
                        qt, kh,
                        dimension_numbers=(((1,), (1,)), ((), ())),
                        preferred_element_type=jnp.float32,
                    ) * SCALE
                    m = jnp.max(s, axis=-1, keepdims=True)
                    w = jnp.exp(s - m) * mask_ref[...].astype(jnp.float32)
                    d = jnp.sum(w, axis=-1, keepdims=True)
                    wb = (w / d).astype(jnp.bfloat16)
                    ctx_ref[pl.ds(row, QTILE),
                            hh * DH:(hh + 1) * DH] = jnp.dot(
                        wb, vh, preferred_element_type=jnp.float32,
                    ).astype(jnp.bfloat16)
                    return 0

                lax.fori_loop(0, NTILES, tile_body, 0)

            contrib = jnp.dot(
                ctx_ref[...], comm_ref[h, 1],
                preferred_element_type=jnp.float32)
            if h == 0:
                out_ref[0] = contrib
            else:
                out_ref[0] = out_ref[0] + contrib

            if h < N_DEV - 1:
                rdma.wait()

    return pl.pallas_call(
        body,
        out_shape=jax.ShapeDtypeStruct((1, SQ, DM), jnp.float32),
        in_specs=[
            pl.BlockSpec(memory_space=pl.ANY),
            pl.BlockSpec(memory_space=pl.ANY),
            pl.BlockSpec(memory_space=pl.ANY),
            pl.BlockSpec(memory_space=pl.ANY),
            pl.BlockSpec(memory_space=pl.ANY),
        ],
        out_specs=pl.BlockSpec(memory_space=pltpu.MemorySpace.VMEM),
        scratch_shapes=[
            pltpu.VMEM((N_DEV, 2, DM, DM), jnp.bfloat16),
            pltpu.VMEM((SQ, DM), jnp.bfloat16),
            pltpu.VMEM((SQ, HPD * DH), jnp.bfloat16),
            pltpu.VMEM((SQ, HPD * DH), jnp.bfloat16),
            pltpu.VMEM((2, SKV, DH), jnp.float32),
            pltpu.VMEM((2, SKV, DH), jnp.float32),
            pltpu.VMEM((DM, DM), jnp.float32),
            pltpu.VMEM((QTILE, SKV), jnp.bfloat16),
            pltpu.SemaphoreType.DMA((N_DEV,)),
            pltpu.SemaphoreType.DMA((N_DEV,)),
            pltpu.SemaphoreType.DMA((2,)),
            pltpu.SemaphoreType.DMA((2,)),
            pltpu.SemaphoreType.DMA,
        ],
        compiler_params=pltpu.CompilerParams(
            collective_id=0, vmem_limit_bytes=63 * 1024 * 1024),
    )(x, Wq, K_ext, V_ext, Wo)
